# Initial kernel scaffold; baseline (speedup 1.0000x reference)
import functools

import jax
import jax.numpy as jnp
from jax import lax
from jax.experimental import pallas as pl
from jax.experimental.pallas import tpu as pltpu
from jax.experimental.pallas import tpu_sc as plsc

N = 10000
E = 320000
R = 3
IN = 128
H = 64
C = 5

NC = 2
NS = 16
L = 16
NW = NC * NS

EK = 128
NCHUNK = E // EK
NT = -(-NCHUNK // NW)

RN = R * N
RPS = RN // NS
ZR = 125

_MESH = plsc.VectorSubcoreMesh(core_axis_name="c", subcore_axis_name="s")


def _sc_aggregate(y_flat, src, dst, etyp):
    @functools.partial(
        pl.kernel,
        out_type=jax.ShapeDtypeStruct((NC, RN, H), jnp.float32),
        mesh=_MESH,
        scratch_types=[
            pltpu.VMEM_SHARED((RN, H), jnp.float32),
            pltpu.VMEM((ZR, H), jnp.float32),
            pltpu.VMEM((EK,), jnp.int32),
            pltpu.VMEM((EK,), jnp.int32),
            pltpu.VMEM((EK,), jnp.int32),
            pltpu.VMEM((EK,), jnp.int32),
            pltpu.VMEM((EK,), jnp.int32),
            pltpu.VMEM((EK, H), jnp.float32),
            pltpu.SemaphoreType.DMA,
        ],
    )
    def k(y_hbm, src_hbm, dst_hbm, typ_hbm, out_hbm,
          acc, zbuf, sbuf, dbuf, tbuf, gbuf, wbuf, rows, sem):
        cid = lax.axis_index("c")
        sid = lax.axis_index("s")
        wid = sid * NC + cid

        @pl.loop(0, ZR)
        def _(i):
            @pl.loop(0, H, step=L)
            def _(j):
                zbuf[i, pl.ds(j, L)] = jnp.zeros((L,), jnp.float32)

        @pl.loop(0, RPS, step=ZR)
        def _(rr):
            pltpu.sync_copy(zbuf, acc.at[pl.ds(sid * RPS + rr, ZR)])

        plsc.subcore_barrier()

        @pl.loop(0, NT)
        def _(t):
            q = wid + t * NW

            @pl.when(q < NCHUNK)
            def _():
                base = q * EK
                pltpu.sync_copy(src_hbm.at[pl.ds(base, EK)], sbuf)
                pltpu.sync_copy(dst_hbm.at[pl.ds(base, EK)], dbuf)
                pltpu.sync_copy(typ_hbm.at[pl.ds(base, EK)], tbuf)

                @pl.loop(0, EK, step=L)
                def _(j):
                    tn = tbuf[pl.ds(j, L)] * N
                    gbuf[pl.ds(j, L)] = tn + sbuf[pl.ds(j, L)]
                    wbuf[pl.ds(j, L)] = tn + dbuf[pl.ds(j, L)]

                pltpu.async_copy(y_hbm.at[gbuf], rows, sem).wait()
                pltpu.sync_copy(rows, acc.at[wbuf], add=True)

        plsc.subcore_barrier()

        pltpu.sync_copy(acc.at[pl.ds(sid * RPS, RPS)],
                        out_hbm.at[cid, pl.ds(sid * RPS, RPS)])

    return k(y_flat, src, dst, etyp)


def _sc_counts(dst, etyp):
    @functools.partial(
        pl.kernel,
        out_type=jax.ShapeDtypeStruct((NC, RN, L), jnp.float32),
        mesh=_MESH,
        scratch_types=[
            pltpu.VMEM_SHARED((RN, L), jnp.float32),
            pltpu.VMEM((ZR, L), jnp.float32),
            pltpu.VMEM((EK,), jnp.int32),
            pltpu.VMEM((EK,), jnp.int32),
            pltpu.VMEM((EK,), jnp.int32),
            pltpu.VMEM((EK, L), jnp.float32),
        ],
    )
    def k(dst_hbm, typ_hbm, out_hbm, acc, zbuf, dbuf, tbuf, wbuf, obuf):
        cid = lax.axis_index("c")
        sid = lax.axis_index("s")
        wid = sid * NC + cid

        onehot = jnp.where(lax.iota(jnp.int32, L) == 0,
                           jnp.float32(1.0), jnp.float32(0.0))
        zero = jnp.zeros((L,), jnp.float32)

        @pl.loop(0, EK)
        def _(i):
            obuf[i, :] = onehot

        @pl.loop(0, ZR)
        def _(i):
            zbuf[i, :] = zero

        @pl.loop(0, RPS, step=ZR)
        def _(rr):
            pltpu.sync_copy(zbuf, acc.at[pl.ds(sid * RPS + rr, ZR)])

        plsc.subcore_barrier()

        @pl.loop(0, NT)
        def _(t):
            q = wid + t * NW

            @pl.when(q < NCHUNK)
            def _():
                base = q * EK
                pltpu.sync_copy(dst_hbm.at[pl.ds(base, EK)], dbuf)
                pltpu.sync_copy(typ_hbm.at[pl.ds(base, EK)], tbuf)

                @pl.loop(0, EK, step=L)
                def _(j):
                    wbuf[pl.ds(j, L)] = (tbuf[pl.ds(j, L)] * N
                                         + dbuf[pl.ds(j, L)])

                pltpu.sync_copy(obuf, acc.at[wbuf], add=True)

        plsc.subcore_barrier()

        pltpu.sync_copy(acc.at[pl.ds(sid * RPS, RPS)],
                        out_hbm.at[cid, pl.ds(sid * RPS, RPS)])

    return k(dst, etyp)


_NB = 2000


def _dot(a, b):
    return jax.lax.dot_general(a, b, (((1,), (0,)), ((), ())),
                               precision=lax.Precision.HIGHEST,
                               preferred_element_type=jnp.float32)


def _dense1(x, W_rel1, W_root1, b1):
    def body(x_ref, wr_ref, wroot_ref, b_ref, y_ref, root_ref):
        xb = x_ref[...]
        for r in range(R):
            y_ref[r] = _dot(xb, wr_ref[r])
        root_ref[...] = _dot(xb, wroot_ref[...]) + b_ref[...]

    grid = (N // _NB,)
    y, root = pl.pallas_call(
        body,
        grid=grid,
        in_specs=[
            pl.BlockSpec((_NB, IN), lambda i: (i, 0)),
            pl.BlockSpec((R, IN, H), lambda i: (0, 0, 0)),
            pl.BlockSpec((IN, H), lambda i: (0, 0)),
            pl.BlockSpec((1, H), lambda i: (0, 0)),
        ],
        out_specs=[
            pl.BlockSpec((R, _NB, H), lambda i: (0, i, 0)),
            pl.BlockSpec((_NB, H), lambda i: (i, 0)),
        ],
        out_shape=[
            jax.ShapeDtypeStruct((R, N, H), jnp.float32),
            jax.ShapeDtypeStruct((N, H), jnp.float32),
        ],
    )(x, W_rel1, W_root1, b1.reshape(1, H))
    return y.reshape(RN, H), root


def _combine(root_blk, p_ref, cnt_ref):
    h = root_blk
    for r in range(R):
        s = p_ref[0, r] + p_ref[1, r]
        cnt = cnt_ref[0, r, :, 0:1] + cnt_ref[1, r, :, 0:1]
        h = h + s * (1.0 / jnp.maximum(cnt, 1.0))
    return jnp.maximum(h, 0.0)


def _dense2(root1, p1, cnt, W_rel2, W_root2, b2):
    def body(root_ref, p_ref, cnt_ref, wr_ref, wroot_ref, b_ref,
             y_ref, root2_ref):
        h = _combine(root_ref[...], p_ref, cnt_ref)
        for r in range(R):
            y_ref[r] = _dot(h, wr_ref[r])
        root2_ref[...] = _dot(h, wroot_ref[...]) + b_ref[...]

    grid = (N // _NB,)
    y, root2 = pl.pallas_call(
        body,
        grid=grid,
        in_specs=[
            pl.BlockSpec((_NB, H), lambda i: (i, 0)),
            pl.BlockSpec((NC, R, _NB, H), lambda i: (0, 0, i, 0)),
            pl.BlockSpec((NC, R, _NB, L), lambda i: (0, 0, i, 0)),
            pl.BlockSpec((R, H, H), lambda i: (0, 0, 0)),
            pl.BlockSpec((H, H), lambda i: (0, 0)),
            pl.BlockSpec((1, H), lambda i: (0, 0)),
        ],
        out_specs=[
            pl.BlockSpec((R, _NB, H), lambda i: (0, i, 0)),
            pl.BlockSpec((_NB, H), lambda i: (i, 0)),
        ],
        out_shape=[
            jax.ShapeDtypeStruct((R, N, H), jnp.float32),
            jax.ShapeDtypeStruct((N, H), jnp.float32),
        ],
    )(root1, p1.reshape(NC, R, N, H), cnt.reshape(NC, R, N, L),
      W_rel2, W_root2, b2.reshape(1, H))
    return y.reshape(RN, H), root2


def _final(root2, p2, cnt, Wc, bc):
    def body(root_ref, p_ref, cnt_ref, wc_ref, bc_ref, out_ref):
        h = _combine(root_ref[...], p_ref, cnt_ref)
        out_ref[...] = _dot(h, wc_ref[...]) + bc_ref[...]

    grid = (N // _NB,)
    return pl.pallas_call(
        body,
        grid=grid,
        in_specs=[
            pl.BlockSpec((_NB, H), lambda i: (i, 0)),
            pl.BlockSpec((NC, R, _NB, H), lambda i: (0, 0, i, 0)),
            pl.BlockSpec((NC, R, _NB, L), lambda i: (0, 0, i, 0)),
            pl.BlockSpec((H, C), lambda i: (0, 0)),
            pl.BlockSpec((1, C), lambda i: (0, 0)),
        ],
        out_specs=pl.BlockSpec((_NB, C), lambda i: (i, 0)),
        out_shape=jax.ShapeDtypeStruct((N, C), jnp.float32),
    )(root2, p2.reshape(NC, R, N, H), cnt.reshape(NC, R, N, L),
      Wc, bc.reshape(1, C))


def kernel(x, edge_index, edge_type, W_rel1, W_root1, b1,
           W_rel2, W_root2, b2, Wc, bc):
    src = edge_index[0]
    dst = edge_index[1]
    cnt = _sc_counts(dst, edge_type)
    y1, root1 = _dense1(x, W_rel1, W_root1, b1)
    p1 = _sc_aggregate(y1, src, dst, edge_type)
    y2, root2 = _dense2(root1, p1, cnt, W_rel2, W_root2, b2)
    p2 = _sc_aggregate(y2, src, dst, edge_type)
    return _final(root2, p2, cnt, Wc, bc)


# speedup vs baseline: 8.2755x; 8.2755x over previous
import functools

import jax
import jax.numpy as jnp
from jax import lax
from jax.experimental import pallas as pl
from jax.experimental.pallas import tpu as pltpu
from jax.experimental.pallas import tpu_sc as plsc

N = 10000
E = 320000
R = 3
IN = 128
H = 64
C = 5

NC = 2
NS = 16
L = 16
NW = NC * NS

EK = 128
NCHUNK = E // EK
NT = -(-NCHUNK // NW)

RN = R * N
DCH = 200
NDC = RN // DCH
DT = -(-NDC // NS)

_MESH = plsc.VectorSubcoreMesh(core_axis_name="c", subcore_axis_name="s")
_SC_PARAMS = pltpu.CompilerParams(use_tc_tiling_on_sc=False)


def _sc_aggregate(y_flat, zeros, src, dst, etyp):
    @functools.partial(
        pl.kernel,
        out_type=jax.ShapeDtypeStruct((NC, RN, H), jnp.float32),
        mesh=_MESH,
        scratch_types=[
            pltpu.VMEM_SHARED((RN, H), jnp.float32),
            pltpu.VMEM((EK,), jnp.int32),
            pltpu.VMEM((EK,), jnp.int32),
            pltpu.VMEM((EK,), jnp.int32),
            pltpu.VMEM((EK,), jnp.int32),
            pltpu.VMEM((EK,), jnp.int32),
            pltpu.VMEM((EK, H), jnp.float32),
            pltpu.SemaphoreType.DMA,
        ],
        compiler_params=_SC_PARAMS,
    )
    def k(y_hbm, z_hbm, src_hbm, dst_hbm, typ_hbm, out_hbm,
          acc, sbuf, dbuf, tbuf, gbuf, wbuf, rows, sem):
        cid = lax.axis_index("c")
        sid = lax.axis_index("s")
        wid = sid * NC + cid

        @pl.loop(0, DT)
        def _(t):
            ci = sid + t * NS

            @pl.when(ci < NDC)
            def _():
                pltpu.sync_copy(z_hbm, acc.at[pl.ds(ci * DCH, DCH)])

        plsc.subcore_barrier()

        @pl.loop(0, NT)
        def _(t):
            q = wid + t * NW

            @pl.when(q < NCHUNK)
            def _():
                base = q * EK
                pltpu.sync_copy(src_hbm.at[pl.ds(base, EK)], sbuf)
                pltpu.sync_copy(dst_hbm.at[pl.ds(base, EK)], dbuf)
                pltpu.sync_copy(typ_hbm.at[pl.ds(base, EK)], tbuf)

                @pl.loop(0, EK, step=L)
                def _(j):
                    tn = tbuf[pl.ds(j, L)] * N
                    gbuf[pl.ds(j, L)] = tn + sbuf[pl.ds(j, L)]
                    wbuf[pl.ds(j, L)] = tn + dbuf[pl.ds(j, L)]

                pltpu.async_copy(y_hbm.at[gbuf], rows, sem).wait()
                pltpu.sync_copy(rows, acc.at[wbuf], add=True)

        plsc.subcore_barrier()

        @pl.loop(0, DT)
        def _(t):
            ci = sid + t * NS

            @pl.when(ci < NDC)
            def _():
                pltpu.sync_copy(acc.at[pl.ds(ci * DCH, DCH)],
                                out_hbm.at[cid, pl.ds(ci * DCH, DCH)])

    return k(y_flat, zeros, src, dst, etyp)


def _sc_counts(zeros, dst, etyp):
    @functools.partial(
        pl.kernel,
        out_type=jax.ShapeDtypeStruct((NC, RN, L), jnp.float32),
        mesh=_MESH,
        scratch_types=[
            pltpu.VMEM_SHARED((RN, L), jnp.float32),
            pltpu.VMEM((EK,), jnp.int32),
            pltpu.VMEM((EK,), jnp.int32),
            pltpu.VMEM((EK,), jnp.int32),
            pltpu.VMEM((EK, L), jnp.float32),
        ],
        compiler_params=_SC_PARAMS,
    )
    def k(z_hbm, dst_hbm, typ_hbm, out_hbm, acc, dbuf, tbuf, wbuf, obuf):
        cid = lax.axis_index("c")
        sid = lax.axis_index("s")
        wid = sid * NC + cid

        onehot = jnp.where(lax.iota(jnp.int32, L) == 0,
                           jnp.float32(1.0), jnp.float32(0.0))

        @pl.loop(0, EK)
        def _(i):
            obuf[i, :] = onehot

        @pl.loop(0, DT)
        def _(t):
            ci = sid + t * NS

            @pl.when(ci < NDC)
            def _():
                pltpu.sync_copy(z_hbm, acc.at[pl.ds(ci * DCH, DCH)])

        plsc.subcore_barrier()

        @pl.loop(0, NT)
        def _(t):
            q = wid + t * NW

            @pl.when(q < NCHUNK)
            def _():
                base = q * EK
                pltpu.sync_copy(dst_hbm.at[pl.ds(base, EK)], dbuf)
                pltpu.sync_copy(typ_hbm.at[pl.ds(base, EK)], tbuf)

                @pl.loop(0, EK, step=L)
                def _(j):
                    wbuf[pl.ds(j, L)] = (tbuf[pl.ds(j, L)] * N
                                         + dbuf[pl.ds(j, L)])

                pltpu.sync_copy(obuf, acc.at[wbuf], add=True)

        plsc.subcore_barrier()

        @pl.loop(0, DT)
        def _(t):
            ci = sid + t * NS

            @pl.when(ci < NDC)
            def _():
                pltpu.sync_copy(acc.at[pl.ds(ci * DCH, DCH)],
                                out_hbm.at[cid, pl.ds(ci * DCH, DCH)])

    return k(zeros, dst, etyp)


_NB = 2000


def _dot(a, b):
    return jax.lax.dot_general(a, b, (((1,), (0,)), ((), ())),
                               precision=lax.Precision.HIGHEST,
                               preferred_element_type=jnp.float32)


def _dense1(x, W_rel1, W_root1, b1):
    def body(x_ref, wr_ref, wroot_ref, b_ref, y_ref, root_ref):
        xb = x_ref[...]
        for r in range(R):
            y_ref[r] = _dot(xb, wr_ref[r])
        root_ref[...] = _dot(xb, wroot_ref[...]) + b_ref[...]

    grid = (N // _NB,)
    y, root = pl.pallas_call(
        body,
        grid=grid,
        in_specs=[
            pl.BlockSpec((_NB, IN), lambda i: (i, 0)),
            pl.BlockSpec((R, IN, H), lambda i: (0, 0, 0)),
            pl.BlockSpec((IN, H), lambda i: (0, 0)),
            pl.BlockSpec((1, H), lambda i: (0, 0)),
        ],
        out_specs=[
            pl.BlockSpec((R, _NB, H), lambda i: (0, i, 0)),
            pl.BlockSpec((_NB, H), lambda i: (i, 0)),
        ],
        out_shape=[
            jax.ShapeDtypeStruct((R, N, H), jnp.float32),
            jax.ShapeDtypeStruct((N, H), jnp.float32),
        ],
    )(x, W_rel1, W_root1, b1.reshape(1, H))
    return y.reshape(RN, H), root


def _combine(root_blk, p_ref, cnt_ref):
    h = root_blk
    for r in range(R):
        s = p_ref[0, r] + p_ref[1, r]
        cnt = cnt_ref[0, r, :, 0:1] + cnt_ref[1, r, :, 0:1]
        h = h + s * (1.0 / jnp.maximum(cnt, 1.0))
    return jnp.maximum(h, 0.0)


def _dense2(root1, p1, cnt, W_rel2, W_root2, b2):
    def body(root_ref, p_ref, cnt_ref, wr_ref, wroot_ref, b_ref,
             y_ref, root2_ref):
        h = _combine(root_ref[...], p_ref, cnt_ref)
        for r in range(R):
            y_ref[r] = _dot(h, wr_ref[r])
        root2_ref[...] = _dot(h, wroot_ref[...]) + b_ref[...]

    grid = (N // _NB,)
    y, root2 = pl.pallas_call(
        body,
        grid=grid,
        in_specs=[
            pl.BlockSpec((_NB, H), lambda i: (i, 0)),
            pl.BlockSpec((NC, R, _NB, H), lambda i: (0, 0, i, 0)),
            pl.BlockSpec((NC, R, _NB, L), lambda i: (0, 0, i, 0)),
            pl.BlockSpec((R, H, H), lambda i: (0, 0, 0)),
            pl.BlockSpec((H, H), lambda i: (0, 0)),
            pl.BlockSpec((1, H), lambda i: (0, 0)),
        ],
        out_specs=[
            pl.BlockSpec((R, _NB, H), lambda i: (0, i, 0)),
            pl.BlockSpec((_NB, H), lambda i: (i, 0)),
        ],
        out_shape=[
            jax.ShapeDtypeStruct((R, N, H), jnp.float32),
            jax.ShapeDtypeStruct((N, H), jnp.float32),
        ],
    )(root1, p1.reshape(NC, R, N, H), cnt.reshape(NC, R, N, L),
      W_rel2, W_root2, b2.reshape(1, H))
    return y.reshape(RN, H), root2


def _final(root2, p2, cnt, Wc, bc):
    def body(root_ref, p_ref, cnt_ref, wc_ref, bc_ref, out_ref):
        h = _combine(root_ref[...], p_ref, cnt_ref)
        out_ref[...] = _dot(h, wc_ref[...]) + bc_ref[...]

    grid = (N // _NB,)
    return pl.pallas_call(
        body,
        grid=grid,
        in_specs=[
            pl.BlockSpec((_NB, H), lambda i: (i, 0)),
            pl.BlockSpec((NC, R, _NB, H), lambda i: (0, 0, i, 0)),
            pl.BlockSpec((NC, R, _NB, L), lambda i: (0, 0, i, 0)),
            pl.BlockSpec((H, C), lambda i: (0, 0)),
            pl.BlockSpec((1, C), lambda i: (0, 0)),
        ],
        out_specs=pl.BlockSpec((_NB, C), lambda i: (i, 0)),
        out_shape=jax.ShapeDtypeStruct((N, C), jnp.float32),
    )(root2, p2.reshape(NC, R, N, H), cnt.reshape(NC, R, N, L),
      Wc, bc.reshape(1, C))


def kernel(x, edge_index, edge_type, W_rel1, W_root1, b1,
           W_rel2, W_root2, b2, Wc, bc):
    src = edge_index[0]
    dst = edge_index[1]
    zeros = jnp.zeros((DCH, H), jnp.float32)
    cnt = _sc_counts(jnp.zeros((DCH, L), jnp.float32), dst, edge_type)
    y1, root1 = _dense1(x, W_rel1, W_root1, b1)
    p1 = _sc_aggregate(y1, zeros, src, dst, edge_type)
    y2, root2 = _dense2(root1, p1, cnt, W_rel2, W_root2, b2)
    p2 = _sc_aggregate(y2, zeros, src, dst, edge_type)
    return _final(root2, p2, cnt, Wc, bc)


# speedup vs baseline: 9.6928x; 1.1713x over previous
import functools

import jax
import jax.numpy as jnp
from jax import lax
from jax.experimental import pallas as pl
from jax.experimental.pallas import tpu as pltpu
from jax.experimental.pallas import tpu_sc as plsc

N = 10000
E = 320000
R = 3
IN = 128
H = 64
C = 5

NC = 2
NS = 16
L = 16
NW = NC * NS

EK = 128
NCHUNK = E // EK
NT = -(-NCHUNK // NW)

RN = R * N
DCH = 200
NDC = RN // DCH
DT = -(-NDC // NS)

_MESH = plsc.VectorSubcoreMesh(core_axis_name="c", subcore_axis_name="s")
_SC_PARAMS = pltpu.CompilerParams(use_tc_tiling_on_sc=False)


def _sc_aggregate(y_flat, zeros, pk):
    @functools.partial(
        pl.kernel,
        out_type=jax.ShapeDtypeStruct((NC, RN, H), jnp.float32),
        mesh=_MESH,
        scratch_types=[
            pltpu.VMEM_SHARED((RN, H), jnp.float32),
            pltpu.VMEM((2, EK), jnp.int32),
            pltpu.VMEM((EK, H), jnp.float32),
            pltpu.SemaphoreType.DMA,
        ],
        compiler_params=_SC_PARAMS,
    )
    def k(y_hbm, z_hbm, pk_hbm, out_hbm, acc, ibuf, rows, sem):
        cid = lax.axis_index("c")
        sid = lax.axis_index("s")
        wid = sid * NC + cid

        @pl.loop(0, DT)
        def _(t):
            ci = sid + t * NS

            @pl.when(ci < NDC)
            def _():
                pltpu.sync_copy(z_hbm, acc.at[pl.ds(ci * DCH, DCH)])

        plsc.subcore_barrier()

        @pl.loop(0, NT)
        def _(t):
            q = wid + t * NW

            @pl.when(q < NCHUNK)
            def _():
                pltpu.sync_copy(pk_hbm.at[q], ibuf)
                pltpu.async_copy(y_hbm.at[ibuf.at[0]], rows, sem).wait()
                pltpu.sync_copy(rows, acc.at[ibuf.at[1]], add=True)

        plsc.subcore_barrier()

        @pl.loop(0, DT)
        def _(t):
            ci = sid + t * NS

            @pl.when(ci < NDC)
            def _():
                pltpu.sync_copy(acc.at[pl.ds(ci * DCH, DCH)],
                                out_hbm.at[cid, pl.ds(ci * DCH, DCH)])

    return k(y_flat, zeros, pk)


def _sc_counts(zeros, src, dst, etyp):
    @functools.partial(
        pl.kernel,
        out_type=[
            jax.ShapeDtypeStruct((NC, RN, L), jnp.float32),
            jax.ShapeDtypeStruct((NCHUNK, 2, EK), jnp.int32),
        ],
        mesh=_MESH,
        scratch_types=[
            pltpu.VMEM_SHARED((RN, L), jnp.float32),
            pltpu.VMEM((EK,), jnp.int32),
            pltpu.VMEM((EK,), jnp.int32),
            pltpu.VMEM((EK,), jnp.int32),
            pltpu.VMEM((2, EK), jnp.int32),
            pltpu.VMEM((EK, L), jnp.float32),
        ],
        compiler_params=_SC_PARAMS,
    )
    def k(z_hbm, src_hbm, dst_hbm, typ_hbm, out_hbm, pk_hbm,
          acc, sbuf, dbuf, tbuf, ibuf, obuf):
        cid = lax.axis_index("c")
        sid = lax.axis_index("s")
        wid = sid * NC + cid

        onehot = jnp.where(lax.iota(jnp.int32, L) == 0,
                           jnp.float32(1.0), jnp.float32(0.0))

        @pl.loop(0, EK)
        def _(i):
            obuf[i, :] = onehot

        @pl.loop(0, DT)
        def _(t):
            ci = sid + t * NS

            @pl.when(ci < NDC)
            def _():
                pltpu.sync_copy(z_hbm, acc.at[pl.ds(ci * DCH, DCH)])

        plsc.subcore_barrier()

        @pl.loop(0, NT)
        def _(t):
            q = wid + t * NW

            @pl.when(q < NCHUNK)
            def _():
                base = q * EK
                pltpu.sync_copy(src_hbm.at[pl.ds(base, EK)], sbuf)
                pltpu.sync_copy(dst_hbm.at[pl.ds(base, EK)], dbuf)
                pltpu.sync_copy(typ_hbm.at[pl.ds(base, EK)], tbuf)

                @pl.loop(0, EK, step=L)
                def _(j):
                    tn = tbuf[pl.ds(j, L)] * N
                    ibuf[0, pl.ds(j, L)] = tn + sbuf[pl.ds(j, L)]
                    ibuf[1, pl.ds(j, L)] = tn + dbuf[pl.ds(j, L)]

                pltpu.sync_copy(ibuf, pk_hbm.at[q])
                pltpu.sync_copy(obuf, acc.at[ibuf.at[1]], add=True)

        plsc.subcore_barrier()

        @pl.loop(0, DT)
        def _(t):
            ci = sid + t * NS

            @pl.when(ci < NDC)
            def _():
                pltpu.sync_copy(acc.at[pl.ds(ci * DCH, DCH)],
                                out_hbm.at[cid, pl.ds(ci * DCH, DCH)])

    return k(zeros, src, dst, etyp)


_NB = 2000


def _dot(a, b):
    return jax.lax.dot_general(a, b, (((1,), (0,)), ((), ())),
                               precision=lax.Precision.HIGHEST,
                               preferred_element_type=jnp.float32)


def _dense1(x, W_rel1, W_root1, b1):
    def body(x_ref, wr_ref, wroot_ref, b_ref, y_ref, root_ref):
        xb = x_ref[...]
        for r in range(R):
            y_ref[r] = _dot(xb, wr_ref[r])
        root_ref[...] = _dot(xb, wroot_ref[...]) + b_ref[...]

    grid = (N // _NB,)
    y, root = pl.pallas_call(
        body,
        grid=grid,
        in_specs=[
            pl.BlockSpec((_NB, IN), lambda i: (i, 0)),
            pl.BlockSpec((R, IN, H), lambda i: (0, 0, 0)),
            pl.BlockSpec((IN, H), lambda i: (0, 0)),
            pl.BlockSpec((1, H), lambda i: (0, 0)),
        ],
        out_specs=[
            pl.BlockSpec((R, _NB, H), lambda i: (0, i, 0)),
            pl.BlockSpec((_NB, H), lambda i: (i, 0)),
        ],
        out_shape=[
            jax.ShapeDtypeStruct((R, N, H), jnp.float32),
            jax.ShapeDtypeStruct((N, H), jnp.float32),
        ],
    )(x, W_rel1, W_root1, b1.reshape(1, H))
    return y.reshape(RN, H), root


def _combine(root_blk, p_ref, cnt_ref):
    h = root_blk
    for r in range(R):
        s = p_ref[0, r] + p_ref[1, r]
        cnt = cnt_ref[0, r, :, 0:1] + cnt_ref[1, r, :, 0:1]
        h = h + s * (1.0 / jnp.maximum(cnt, 1.0))
    return jnp.maximum(h, 0.0)


def _dense2(root1, p1, cnt, W_rel2, W_root2, b2):
    def body(root_ref, p_ref, cnt_ref, wr_ref, wroot_ref, b_ref,
             y_ref, root2_ref):
        h = _combine(root_ref[...], p_ref, cnt_ref)
        for r in range(R):
            y_ref[r] = _dot(h, wr_ref[r])
        root2_ref[...] = _dot(h, wroot_ref[...]) + b_ref[...]

    grid = (N // _NB,)
    y, root2 = pl.pallas_call(
        body,
        grid=grid,
        in_specs=[
            pl.BlockSpec((_NB, H), lambda i: (i, 0)),
            pl.BlockSpec((NC, R, _NB, H), lambda i: (0, 0, i, 0)),
            pl.BlockSpec((NC, R, _NB, L), lambda i: (0, 0, i, 0)),
            pl.BlockSpec((R, H, H), lambda i: (0, 0, 0)),
            pl.BlockSpec((H, H), lambda i: (0, 0)),
            pl.BlockSpec((1, H), lambda i: (0, 0)),
        ],
        out_specs=[
            pl.BlockSpec((R, _NB, H), lambda i: (0, i, 0)),
            pl.BlockSpec((_NB, H), lambda i: (i, 0)),
        ],
        out_shape=[
            jax.ShapeDtypeStruct((R, N, H), jnp.float32),
            jax.ShapeDtypeStruct((N, H), jnp.float32),
        ],
    )(root1, p1.reshape(NC, R, N, H), cnt.reshape(NC, R, N, L),
      W_rel2, W_root2, b2.reshape(1, H))
    return y.reshape(RN, H), root2


def _final(root2, p2, cnt, Wc, bc):
    def body(root_ref, p_ref, cnt_ref, wc_ref, bc_ref, out_ref):
        h = _combine(root_ref[...], p_ref, cnt_ref)
        out_ref[...] = _dot(h, wc_ref[...]) + bc_ref[...]

    grid = (N // _NB,)
    return pl.pallas_call(
        body,
        grid=grid,
        in_specs=[
            pl.BlockSpec((_NB, H), lambda i: (i, 0)),
            pl.BlockSpec((NC, R, _NB, H), lambda i: (0, 0, i, 0)),
            pl.BlockSpec((NC, R, _NB, L), lambda i: (0, 0, i, 0)),
            pl.BlockSpec((H, C), lambda i: (0, 0)),
            pl.BlockSpec((1, C), lambda i: (0, 0)),
        ],
        out_specs=pl.BlockSpec((_NB, C), lambda i: (i, 0)),
        out_shape=jax.ShapeDtypeStruct((N, C), jnp.float32),
    )(root2, p2.reshape(NC, R, N, H), cnt.reshape(NC, R, N, L),
      Wc, bc.reshape(1, C))


def kernel(x, edge_index, edge_type, W_rel1, W_root1, b1,
           W_rel2, W_root2, b2, Wc, bc):
    src = edge_index[0]
    dst = edge_index[1]
    zeros = jnp.zeros((DCH, H), jnp.float32)
    cnt, pk = _sc_counts(jnp.zeros((DCH, L), jnp.float32),
                         src, dst, edge_type)
    y1, root1 = _dense1(x, W_rel1, W_root1, b1)
    p1 = _sc_aggregate(y1, zeros, pk)
    y2, root2 = _dense2(root1, p1, cnt, W_rel2, W_root2, b2)
    p2 = _sc_aggregate(y2, zeros, pk)
    return _final(root2, p2, cnt, Wc, bc)
